# Initial kernel scaffold; baseline (speedup 1.0000x reference)
import jax
import jax.numpy as jnp
from jax.experimental import pallas as pl


def kernel(input, table):
    raise NotImplementedError("write your pallas kernel here")


# speedup vs baseline: 2.9034x; 2.9034x over previous
import functools

import jax
import jax.numpy as jnp
from jax import lax
from jax.experimental import pallas as pl
from jax.experimental.pallas import tpu as pltpu
from jax.experimental.pallas import tpu_sc as plsc

EMB_DIM = 128

NC = 2
NS = 16
NW = NC * NS

CHUNK = 128


@functools.partial(jax.jit, static_argnums=(1,))
def _gather_rows(idx_flat, n_rows, table):
    b_per_w = n_rows // NW
    n_iter = b_per_w // CHUNK
    mesh = plsc.VectorSubcoreMesh(core_axis_name="c", subcore_axis_name="s")

    @functools.partial(
        pl.kernel,
        mesh=mesh,
        out_type=jax.ShapeDtypeStruct((n_rows, EMB_DIM), jnp.float32),
        scratch_types=[
            pltpu.VMEM((CHUNK,), jnp.int32),
            pltpu.VMEM((CHUNK, EMB_DIM), jnp.float32),
            pltpu.SemaphoreType.DMA,
        ],
    )
    def k(idx_hbm, table_hbm, out_hbm, idx_v, rows_v, sem):
        wid = lax.axis_index("s") * NC + lax.axis_index("c")
        base0 = wid * b_per_w

        def body(g, carry):
            base = base0 + g * CHUNK
            pltpu.sync_copy(idx_hbm.at[pl.ds(base, CHUNK)], idx_v)
            pltpu.async_copy(table_hbm.at[idx_v], rows_v, sem).wait()
            pltpu.sync_copy(rows_v, out_hbm.at[pl.ds(base, CHUNK)])
            return carry

        lax.fori_loop(0, n_iter, body, 0)

    return k(idx_flat, table)


def kernel(input, table):
    b, f = input.shape
    flat = input.reshape(b * f).astype(jnp.int32)
    out = _gather_rows(flat, b * f, table)
    return out.reshape(b, f, EMB_DIM)


# speedup vs baseline: 3.6266x; 1.2491x over previous
import functools

import jax
import jax.numpy as jnp
from jax import lax
from jax.experimental import pallas as pl
from jax.experimental.pallas import tpu as pltpu
from jax.experimental.pallas import tpu_sc as plsc

EMB_DIM = 128

NC = 2
NS = 16
NW = NC * NS

CHUNK = 128
NBUF = 4


@functools.partial(jax.jit, static_argnums=(1,))
def _gather_rows(idx2d, n_rows, table):
    b_per_w = n_rows // NW
    n_chunks = b_per_w // CHUNK
    mesh = plsc.VectorSubcoreMesh(core_axis_name="c", subcore_axis_name="s")

    @functools.partial(
        pl.kernel,
        mesh=mesh,
        out_type=jax.ShapeDtypeStruct((n_rows, EMB_DIM), jnp.float32),
        scratch_types=[
            pltpu.VMEM((n_chunks, CHUNK), jnp.int32),
            pltpu.VMEM((CHUNK, EMB_DIM), jnp.float32),
            pltpu.VMEM((CHUNK, EMB_DIM), jnp.float32),
            pltpu.VMEM((CHUNK, EMB_DIM), jnp.float32),
            pltpu.VMEM((CHUNK, EMB_DIM), jnp.float32),
            pltpu.SemaphoreType.DMA,
            pltpu.SemaphoreType.DMA,
            pltpu.SemaphoreType.DMA,
            pltpu.SemaphoreType.DMA,
        ],
    )
    def k(idx_hbm, table_hbm, out_hbm, idx_v, b0, b1, b2, b3, s0, s1, s2, s3):
        bufs = (b0, b1, b2, b3)
        sems = (s0, s1, s2, s3)
        wid = lax.axis_index("s") * NC + lax.axis_index("c")
        row0 = wid * b_per_w

        pltpu.sync_copy(idx_hbm.at[wid], idx_v)

        def gather(c, b):
            pltpu.async_copy(table_hbm.at[idx_v.at[c]], bufs[b], sems[b])

        for c in range(NBUF - 1):
            gather(c, c)

        def body(g, carry):
            for b in range(NBUF):
                c = g * NBUF + b
                pltpu.make_async_copy(table_hbm.at[idx_v.at[c]], bufs[b],
                                      sems[b]).wait()

                @pl.when(c + NBUF - 1 < n_chunks)
                def _():
                    gather(c + NBUF - 1, (b + NBUF - 1) % NBUF)

                pltpu.sync_copy(bufs[b],
                                out_hbm.at[pl.ds(row0 + c * CHUNK, CHUNK)])
            return carry

        lax.fori_loop(0, n_chunks // NBUF, body, 0)

    return k(idx2d, table)


def kernel(input, table):
    b, f = input.shape
    n = b * f
    idx3d = input.reshape(NW, n // (NW * CHUNK), CHUNK).astype(jnp.int32)
    out = _gather_rows(idx3d, n, table)
    return out.reshape(b, f, EMB_DIM)


# speedup vs baseline: 6.2477x; 1.7228x over previous
import functools

import jax
import jax.numpy as jnp
from jax import lax
from jax.experimental import pallas as pl
from jax.experimental.pallas import tpu as pltpu
from jax.experimental.pallas import tpu_sc as plsc

EMB_DIM = 128

NC = 2
NS = 16
NW = NC * NS

NBUF = 4


@jax.jit
def _gather_rows(idx, table):
    batch, fields = idx.shape
    s_per_w = batch // NW
    mesh = plsc.VectorSubcoreMesh(core_axis_name="c", subcore_axis_name="s")

    @functools.partial(
        pl.kernel,
        mesh=mesh,
        out_type=jax.ShapeDtypeStruct((batch, fields, EMB_DIM), jnp.float32),
        scratch_types=[
            pltpu.VMEM((s_per_w, fields), jnp.int32),
            pltpu.VMEM((fields, EMB_DIM), jnp.float32),
            pltpu.VMEM((fields, EMB_DIM), jnp.float32),
            pltpu.VMEM((fields, EMB_DIM), jnp.float32),
            pltpu.VMEM((fields, EMB_DIM), jnp.float32),
            pltpu.SemaphoreType.DMA,
            pltpu.SemaphoreType.DMA,
            pltpu.SemaphoreType.DMA,
            pltpu.SemaphoreType.DMA,
        ],
    )
    def k(idx_hbm, table_hbm, out_hbm, idx_v, b0, b1, b2, b3, s0, s1, s2, s3):
        bufs = (b0, b1, b2, b3)
        sems = (s0, s1, s2, s3)
        wid = lax.axis_index("s") * NC + lax.axis_index("c")
        sample0 = wid * s_per_w

        pltpu.sync_copy(idx_hbm.at[pl.ds(sample0, s_per_w)], idx_v)

        def gather(c, b):
            pltpu.async_copy(table_hbm.at[idx_v.at[c]], bufs[b], sems[b])

        for c in range(NBUF - 1):
            gather(c, c)

        def body(g, carry):
            for b in range(NBUF):
                c = g * NBUF + b
                pltpu.make_async_copy(table_hbm.at[idx_v.at[c]], bufs[b],
                                      sems[b]).wait()

                @pl.when(c + NBUF - 1 < s_per_w)
                def _():
                    gather(c + NBUF - 1, (b + NBUF - 1) % NBUF)

                pltpu.sync_copy(bufs[b], out_hbm.at[sample0 + c])
            return carry

        lax.fori_loop(0, s_per_w // NBUF, body, 0)

    return k(idx, table)


def kernel(input, table):
    return _gather_rows(input.astype(jnp.int32), table)
